# baseline (device time: 25653 ns/iter reference)
import jax
import jax.numpy as jnp
from jax import lax
from jax.experimental import pallas as pl
from jax.experimental.pallas import tpu as pltpu

N_DEV = 4
HALF = 128
BF = jnp.bfloat16
F32 = jnp.float32


def kernel(x, Win0, Wout0, Win1, Wout1, Win2, Wout2):
    b, d = x.shape
    hid = Win0.shape[1]
    rows_out = b // N_DEV

    def body(x_hbm, win0_hbm, wout0_hbm, win1_hbm, wout1_hbm, win2_hbm,
             wout2_hbm, out_hbm, xv, w0i, w0o, w1i, w1o, w2i, w2o, out_v,
             comm_ref, p_ref, h_ref, rs_ref, send_sems, recv_sems, load_sems):
        my_pos = lax.axis_index("i")
        partner_a = jnp.bitwise_xor(my_pos, 1)
        partner_b = jnp.bitwise_xor(my_pos, 3)

        loads = []
        for k, (src, dst) in enumerate([
            (x_hbm, xv), (win0_hbm, w0i), (wout0_hbm, w0o),
            (win1_hbm, w1i), (wout1_hbm, w1o),
            (win2_hbm, w2i), (wout2_hbm, w2o),
        ]):
            cp = pltpu.make_async_copy(src, dst, load_sems.at[k])
            cp.start()
            loads.append(cp)

        barrier_sem = pltpu.get_barrier_semaphore()
        for nbr in (partner_a, partner_b):
            pl.semaphore_signal(
                barrier_sem, inc=1,
                device_id=(nbr,), device_id_type=pl.DeviceIdType.MESH,
            )
        pl.semaphore_wait(barrier_sem, 2)

        def exchange(send_slot, recv_slot, sem, partner, value):
            comm_ref[send_slot] = value.astype(BF)
            rdma = pltpu.make_async_remote_copy(
                src_ref=comm_ref.at[send_slot],
                dst_ref=comm_ref.at[recv_slot],
                send_sem=send_sems.at[sem],
                recv_sem=recv_sems.at[sem],
                device_id=(partner,),
                device_id_type=pl.DeviceIdType.MESH,
            )
            rdma.start()
            return rdma

        def all_reduce(pa, pb, l, flight1, flight2):
            s = l * 8
            ra = exchange(s + 0, s + 1, l * 4 + 0, partner_a, pa)
            rb = exchange(s + 2, s + 3, l * 4 + 1, partner_b, pb)
            f1 = flight1()
            ra.wait()
            acc_a = pa + comm_ref[s + 1].astype(F32)
            ra2 = exchange(s + 4, s + 5, l * 4 + 2, partner_b, acc_a)
            rb.wait()
            acc_b = pb + comm_ref[s + 3].astype(F32)
            rb2 = exchange(s + 6, s + 7, l * 4 + 3, partner_a, acc_b)
            f2 = flight2()
            ra2.wait()
            sum_a = acc_a + comm_ref[s + 5].astype(F32)
            rb2.wait()
            sum_b = acc_b + comm_ref[s + 7].astype(F32)
            return sum_a, sum_b, f1, f2

        def load_cast(idx, ref):
            loads[idx].wait()
            return ref[:, :].astype(BF)

        xb = load_cast(0, xv)
        win0b = load_cast(1, w0i)
        h = jnp.maximum(jnp.dot(xb, win0b, preferred_element_type=F32), 0.0)
        hb = h.astype(BF)
        wout0b = load_cast(2, w0o)
        pa = jnp.dot(hb, wout0b[:, :HALF], preferred_element_type=F32)
        pb = jnp.dot(hb, wout0b[:, HALF:], preferred_element_type=F32)
        sum_a, sum_b, win1b, wout1b = all_reduce(
            pa, pb, 0,
            lambda: load_cast(3, w1i),
            lambda: load_cast(4, w1o),
        )

        h = jnp.maximum(
            jnp.dot(sum_a.astype(BF), win1b[:HALF, :], preferred_element_type=F32)
            + jnp.dot(sum_b.astype(BF), win1b[HALF:, :], preferred_element_type=F32),
            0.0,
        )
        hb = h.astype(BF)
        pa = jnp.dot(hb, wout1b[:, :HALF], preferred_element_type=F32)
        pb = jnp.dot(hb, wout1b[:, HALF:], preferred_element_type=F32)
        sum_a, sum_b, win2b, wout2b = all_reduce(
            pa, pb, 1,
            lambda: load_cast(5, w2i),
            lambda: load_cast(6, w2o),
        )

        h_ref[:, :] = jnp.maximum(
            jnp.dot(sum_a.astype(BF), win2b[:HALF, :], preferred_element_type=F32)
            + jnp.dot(sum_b.astype(BF), win2b[HALF:, :], preferred_element_type=F32),
            0.0,
        ).astype(BF)

        rdmas = []
        for i in (2, 1, 3):
            t = lax.rem(my_pos + i, N_DEV)
            rows = pl.ds(t * rows_out, rows_out)
            hc = h_ref[rows, :]
            p_ref[rows, :] = jnp.dot(
                hc, wout2b, preferred_element_type=F32
            ).astype(BF)
            rdma = pltpu.make_async_remote_copy(
                src_ref=p_ref.at[rows, :],
                dst_ref=rs_ref.at[3 - i],
                send_sem=send_sems.at[8 + (i - 1)],
                recv_sem=recv_sems.at[8 + (3 - i)],
                device_id=(t,),
                device_id_type=pl.DeviceIdType.MESH,
            )
            rdma.start()
            rdmas.append(rdma)
        my_rows = pl.ds(my_pos * rows_out, rows_out)
        pm = jnp.dot(h_ref[my_rows, :], wout2b, preferred_element_type=F32)
        for rdma in rdmas:
            rdma.wait_send()
            rdma.wait_recv()

        out_v[:, :] = (
            pm
            + rs_ref[0].astype(F32)
            + rs_ref[1].astype(F32)
            + rs_ref[2].astype(F32)
        )
        out_cp = pltpu.make_async_copy(out_v, out_hbm, load_sems.at[7])
        out_cp.start()
        out_cp.wait()

    return pl.pallas_call(
        body,
        out_shape=jax.ShapeDtypeStruct((rows_out, d), F32),
        in_specs=[pl.BlockSpec(memory_space=pl.ANY)] * 7,
        out_specs=pl.BlockSpec(memory_space=pl.ANY),
        scratch_shapes=[
            pltpu.VMEM((b, d), F32),
            pltpu.VMEM((b, hid), F32),
            pltpu.VMEM((hid, d), F32),
            pltpu.VMEM((b, hid), F32),
            pltpu.VMEM((hid, d), F32),
            pltpu.VMEM((b, hid), F32),
            pltpu.VMEM((hid, d), F32),
            pltpu.VMEM((rows_out, d), F32),
            pltpu.VMEM((16, b, HALF), BF),
            pltpu.VMEM((b, d), BF),
            pltpu.VMEM((b, hid), BF),
            pltpu.VMEM((3, rows_out, d), BF),
            pltpu.SemaphoreType.DMA((11,)),
            pltpu.SemaphoreType.DMA((11,)),
            pltpu.SemaphoreType.DMA((8,)),
        ],
        compiler_params=pltpu.CompilerParams(collective_id=0),
    )(x, Win0, Wout0, Win1, Wout1, Win2, Wout2)


# device time: 18456 ns/iter; 1.3900x vs baseline; 1.3900x over previous
import jax
import jax.numpy as jnp
from jax import lax
from jax.experimental import pallas as pl
from jax.experimental.pallas import tpu as pltpu

N_DEV = 4
HALF = 128
BF = jnp.bfloat16
F32 = jnp.float32


def kernel(x, Win0, Wout0, Win1, Wout1, Win2, Wout2):
    b, d = x.shape
    hid = Win0.shape[1]
    rows_out = b // N_DEV

    def body(x_hbm, win0_hbm, wout0_hbm, win1_hbm, wout1_hbm, win2_hbm,
             wout2_hbm, out_hbm, xv, w0i, w0o, w1i, w1o, w2i, w2o, out_v,
             comm_ref, p_ref, h_ref, rs_ref, send_sems, recv_sems, load_sems):
        my_pos = lax.axis_index("i")
        partner_a = jnp.bitwise_xor(my_pos, 1)
        partner_b = jnp.bitwise_xor(my_pos, 3)

        loads = []
        for k, (src, dst) in enumerate([
            (x_hbm, xv), (win0_hbm, w0i), (wout0_hbm, w0o),
            (win1_hbm, w1i), (wout1_hbm, w1o),
            (win2_hbm, w2i), (wout2_hbm, w2o),
        ]):
            cp = pltpu.make_async_copy(src, dst, load_sems.at[k])
            cp.start()
            loads.append(cp)

        barrier_sem = pltpu.get_barrier_semaphore()
        for nbr in (partner_a, partner_b):
            pl.semaphore_signal(
                barrier_sem, inc=1,
                device_id=(nbr,), device_id_type=pl.DeviceIdType.MESH,
            )
        pl.semaphore_wait(barrier_sem, 2)

        def exchange(send_slot, recv_slot, sem, partner, value):
            comm_ref[send_slot] = value.astype(BF)
            rdma = pltpu.make_async_remote_copy(
                src_ref=comm_ref.at[send_slot],
                dst_ref=comm_ref.at[recv_slot],
                send_sem=send_sems.at[sem],
                recv_sem=recv_sems.at[sem],
                device_id=(partner,),
                device_id_type=pl.DeviceIdType.MESH,
            )
            rdma.start()
            return rdma

        def all_reduce(pa, pb, l, flight1, flight2):
            s = l * 8
            ra = exchange(s + 0, s + 1, l * 4 + 0, partner_a, pa)
            rb = exchange(s + 2, s + 3, l * 4 + 1, partner_b, pb)
            f1 = flight1()
            ra.wait()
            acc_a = pa + comm_ref[s + 1].astype(F32)
            ra2 = exchange(s + 4, s + 5, l * 4 + 2, partner_b, acc_a)
            rb.wait()
            acc_b = pb + comm_ref[s + 3].astype(F32)
            rb2 = exchange(s + 6, s + 7, l * 4 + 3, partner_a, acc_b)
            f2 = flight2()
            ra2.wait()
            sum_a = acc_a + comm_ref[s + 5].astype(F32)
            rb2.wait()
            sum_b = acc_b + comm_ref[s + 7].astype(F32)
            return sum_a, sum_b, f1, f2

        def load_cast(idx, ref):
            loads[idx].wait()
            return ref[:, :].astype(BF)

        xb = load_cast(0, xv)
        win0b = load_cast(1, w0i)
        h = jnp.maximum(jnp.dot(xb, win0b, preferred_element_type=F32), 0.0)
        hb = h.astype(BF)
        wout0b = load_cast(2, w0o)
        pa = jnp.dot(hb, wout0b[:, :HALF], preferred_element_type=F32)
        pb = jnp.dot(hb, wout0b[:, HALF:], preferred_element_type=F32)
        sum_a, sum_b, win1b, wout1b = all_reduce(
            pa, pb, 0,
            lambda: load_cast(3, w1i),
            lambda: load_cast(4, w1o),
        )

        h = jnp.maximum(
            jnp.dot(sum_a.astype(BF), win1b[:HALF, :], preferred_element_type=F32)
            + jnp.dot(sum_b.astype(BF), win1b[HALF:, :], preferred_element_type=F32),
            0.0,
        )
        hb = h.astype(BF)
        pa = jnp.dot(hb, wout1b[:, :HALF], preferred_element_type=F32)
        pb = jnp.dot(hb, wout1b[:, HALF:], preferred_element_type=F32)
        sum_a, sum_b, win2b, wout2b = all_reduce(
            pa, pb, 1,
            lambda: load_cast(5, w2i),
            lambda: load_cast(6, w2o),
        )

        h_ref[:, :] = jnp.maximum(
            jnp.dot(sum_a.astype(BF), win2b[:HALF, :], preferred_element_type=F32)
            + jnp.dot(sum_b.astype(BF), win2b[HALF:, :], preferred_element_type=F32),
            0.0,
        ).astype(BF)

        rdmas = []
        for i in (2, 1, 3):
            t = lax.rem(my_pos + i, N_DEV)
            rows = pl.ds(t * rows_out, rows_out)
            hc = h_ref[rows, :]
            p_ref[rows, :] = jnp.dot(
                hc, wout2b, preferred_element_type=F32
            ).astype(BF)
            rdma = pltpu.make_async_remote_copy(
                src_ref=p_ref.at[rows, :],
                dst_ref=rs_ref.at[3 - i],
                send_sem=send_sems.at[8 + (i - 1)],
                recv_sem=recv_sems.at[8 + (3 - i)],
                device_id=(t,),
                device_id_type=pl.DeviceIdType.MESH,
            )
            rdma.start()
            rdmas.append(rdma)
        my_rows = pl.ds(my_pos * rows_out, rows_out)
        pm = jnp.dot(h_ref[my_rows, :], wout2b, preferred_element_type=F32)
        for rdma in rdmas:
            rdma.wait_send()
            rdma.wait_recv()

        out_v[:, :] = (
            pm
            + rs_ref[0].astype(F32)
            + rs_ref[1].astype(F32)
            + rs_ref[2].astype(F32)
        )
        out_cp = pltpu.make_async_copy(out_v, out_hbm, load_sems.at[7])
        out_cp.start()
        out_cp.wait()

    return pl.pallas_call(
        body,
        out_shape=jax.ShapeDtypeStruct((rows_out, d), F32),
        in_specs=[pl.BlockSpec(memory_space=pl.ANY)] * 7,
        out_specs=pl.BlockSpec(memory_space=pl.ANY),
        scratch_shapes=[
            pltpu.VMEM((b, d), F32),
            pltpu.VMEM((b, hid), F32),
            pltpu.VMEM((hid, d), F32),
            pltpu.VMEM((b, hid), F32),
            pltpu.VMEM((hid, d), F32),
            pltpu.VMEM((b, hid), F32),
            pltpu.VMEM((hid, d), F32),
            pltpu.VMEM((rows_out, d), F32),
            pltpu.VMEM((16, b, HALF), BF),
            pltpu.VMEM((b, d), BF),
            pltpu.VMEM((b, hid), BF),
            pltpu.VMEM((3, rows_out, d), BF),
            pltpu.SemaphoreType.DMA((11,)),
            pltpu.SemaphoreType.DMA((11,)),
            pltpu.SemaphoreType.DMA((8,)),
        ],
        compiler_params=pltpu.CompilerParams(collective_id=0),
    )(*(
        pltpu.with_memory_space_constraint(a, pltpu.MemorySpace.HBM)
        for a in (x, Win0, Wout0, Win1, Wout1, Win2, Wout2)
    ))


# device time: 18359 ns/iter; 1.3973x vs baseline; 1.0053x over previous
import jax
import jax.numpy as jnp
from jax import lax
from jax.experimental import pallas as pl
from jax.experimental.pallas import tpu as pltpu

N_DEV = 4
HALF = 128
BF = jnp.bfloat16
F32 = jnp.float32


def kernel(x, Win0, Wout0, Win1, Wout1, Win2, Wout2):
    b, d = x.shape
    hid = Win0.shape[1]
    rows_out = b // N_DEV

    def body(x_hbm, win0_hbm, wout0_hbm, win1_hbm, wout1_hbm, win2_hbm,
             wout2_hbm, out_ref, xv, w0i, w0o, w1i, w1o, w2i, w2o,
             comm_ref, p_ref, h_ref, rs_ref, send_sems, recv_sems, load_sems):
        my_pos = lax.axis_index("i")
        partner_a = jnp.bitwise_xor(my_pos, 1)
        partner_b = jnp.bitwise_xor(my_pos, 3)

        loads = []
        for k, (src, dst) in enumerate([
            (x_hbm, xv), (win0_hbm, w0i), (wout0_hbm, w0o),
            (win1_hbm, w1i), (wout1_hbm, w1o),
            (win2_hbm, w2i), (wout2_hbm, w2o),
        ]):
            cp = pltpu.make_async_copy(src, dst, load_sems.at[k])
            cp.start()
            loads.append(cp)

        barrier_sem = pltpu.get_barrier_semaphore()
        for nbr in (partner_a, partner_b):
            pl.semaphore_signal(
                barrier_sem, inc=1,
                device_id=(nbr,), device_id_type=pl.DeviceIdType.MESH,
            )
        pl.semaphore_wait(barrier_sem, 2)

        def exchange(send_slot, recv_slot, sem, partner, value):
            comm_ref[send_slot] = value.astype(BF)
            rdma = pltpu.make_async_remote_copy(
                src_ref=comm_ref.at[send_slot],
                dst_ref=comm_ref.at[recv_slot],
                send_sem=send_sems.at[sem],
                recv_sem=recv_sems.at[sem],
                device_id=(partner,),
                device_id_type=pl.DeviceIdType.MESH,
            )
            rdma.start()
            return rdma

        def all_reduce(pa, pb, l, flight1, flight2):
            s = l * 8
            ra = exchange(s + 0, s + 1, l * 4 + 0, partner_a, pa)
            rb = exchange(s + 2, s + 3, l * 4 + 1, partner_b, pb)
            f1 = flight1()
            ra.wait()
            acc_a = pa + comm_ref[s + 1].astype(F32)
            ra2 = exchange(s + 4, s + 5, l * 4 + 2, partner_b, acc_a)
            rb.wait()
            acc_b = pb + comm_ref[s + 3].astype(F32)
            rb2 = exchange(s + 6, s + 7, l * 4 + 3, partner_a, acc_b)
            f2 = flight2()
            ra2.wait()
            sum_a = acc_a + comm_ref[s + 5].astype(F32)
            rb2.wait()
            sum_b = acc_b + comm_ref[s + 7].astype(F32)
            return sum_a, sum_b, f1, f2

        def load_cast(idx, ref):
            loads[idx].wait()
            return ref[:, :].astype(BF)

        xb = load_cast(0, xv)
        win0b = load_cast(1, w0i)
        h = jnp.maximum(jnp.dot(xb, win0b, preferred_element_type=F32), 0.0)
        hb = h.astype(BF)
        wout0b = load_cast(2, w0o)
        pa = jnp.dot(hb, wout0b[:, :HALF], preferred_element_type=F32)
        pb = jnp.dot(hb, wout0b[:, HALF:], preferred_element_type=F32)
        sum_a, sum_b, win1b, wout1b = all_reduce(
            pa, pb, 0,
            lambda: load_cast(3, w1i),
            lambda: load_cast(4, w1o),
        )

        h = jnp.maximum(
            jnp.dot(sum_a.astype(BF), win1b[:HALF, :], preferred_element_type=F32)
            + jnp.dot(sum_b.astype(BF), win1b[HALF:, :], preferred_element_type=F32),
            0.0,
        )
        hb = h.astype(BF)
        pa = jnp.dot(hb, wout1b[:, :HALF], preferred_element_type=F32)
        pb = jnp.dot(hb, wout1b[:, HALF:], preferred_element_type=F32)
        sum_a, sum_b, win2b, wout2b = all_reduce(
            pa, pb, 1,
            lambda: load_cast(5, w2i),
            lambda: load_cast(6, w2o),
        )

        h_ref[:, :] = jnp.maximum(
            jnp.dot(sum_a.astype(BF), win2b[:HALF, :], preferred_element_type=F32)
            + jnp.dot(sum_b.astype(BF), win2b[HALF:, :], preferred_element_type=F32),
            0.0,
        ).astype(BF)

        rdmas = []
        for i in (2, 1, 3):
            t = lax.rem(my_pos + i, N_DEV)
            rows = pl.ds(t * rows_out, rows_out)
            hc = h_ref[rows, :]
            p_ref[rows, :] = jnp.dot(
                hc, wout2b, preferred_element_type=F32
            ).astype(BF)
            rdma = pltpu.make_async_remote_copy(
                src_ref=p_ref.at[rows, :],
                dst_ref=rs_ref.at[3 - i],
                send_sem=send_sems.at[8 + (i - 1)],
                recv_sem=recv_sems.at[8 + (3 - i)],
                device_id=(t,),
                device_id_type=pl.DeviceIdType.MESH,
            )
            rdma.start()
            rdmas.append(rdma)
        my_rows = pl.ds(my_pos * rows_out, rows_out)
        pm = jnp.dot(h_ref[my_rows, :], wout2b, preferred_element_type=F32)
        for rdma in rdmas:
            rdma.wait_send()
            rdma.wait_recv()

        out_ref[:, :] = (
            pm
            + rs_ref[0].astype(F32)
            + rs_ref[1].astype(F32)
            + rs_ref[2].astype(F32)
        )

    return pl.pallas_call(
        body,
        out_shape=jax.ShapeDtypeStruct((rows_out, d), F32),
        in_specs=[pl.BlockSpec(memory_space=pl.ANY)] * 7,
        out_specs=pl.BlockSpec(memory_space=pltpu.VMEM),
        scratch_shapes=[
            pltpu.VMEM((b, d), F32),
            pltpu.VMEM((b, hid), F32),
            pltpu.VMEM((hid, d), F32),
            pltpu.VMEM((b, hid), F32),
            pltpu.VMEM((hid, d), F32),
            pltpu.VMEM((b, hid), F32),
            pltpu.VMEM((hid, d), F32),
            pltpu.VMEM((16, b, HALF), BF),
            pltpu.VMEM((b, d), BF),
            pltpu.VMEM((b, hid), BF),
            pltpu.VMEM((3, rows_out, d), BF),
            pltpu.SemaphoreType.DMA((11,)),
            pltpu.SemaphoreType.DMA((11,)),
            pltpu.SemaphoreType.DMA((8,)),
        ],
        compiler_params=pltpu.CompilerParams(collective_id=0),
    )(*(
        pltpu.with_memory_space_constraint(a, pltpu.MemorySpace.HBM)
        for a in (x, Win0, Wout0, Win1, Wout1, Win2, Wout2)
    ))
